# Initial kernel scaffold; baseline (speedup 1.0000x reference)
import jax
import jax.numpy as jnp
from jax.experimental import pallas as pl


def kernel(x, signal_belongings, W, b):
    raise NotImplementedError("write your pallas kernel here")


# speedup vs baseline: 3.4356x; 3.4356x over previous
import functools

import jax
import jax.numpy as jnp
from jax import lax
from jax.experimental import pallas as pl
from jax.experimental.pallas import tpu as pltpu
from jax.experimental.pallas import tpu_sc as plsc

N_ROWS = 160000
D_IN = 256
D_OUT = 512
N_SEG = 10000

NC = 2
NS = 16
HALF = D_IN // NC

CHUNK = 80
ROWS_PER_TILE = N_ROWS // NS
N_CHUNKS = ROWS_PER_TILE // CHUNK
NSEG_PAD = 10240
SEG_PER_TILE = NSEG_PAD // NS
ZCH = 128
CNT_W = 16


def _sc_body(x_hbm, ids_hbm, acc_hbm, cnt_hbm,
             acc_s, cnt_s, rows_v, idx_v, ones_v, zbuf, cntbuf):
    c = lax.axis_index("c")
    s = lax.axis_index("s")

    def _zrow(i, _):
        def _zcol(j, _):
            zbuf[i, pl.ds(j * 16, 16)] = jnp.zeros((16,), jnp.float32)
            return 0
        return lax.fori_loop(0, HALF // 16, _zcol, 0)
    lax.fori_loop(0, ZCH, _zrow, 0)

    def _orow(i, _):
        ones_v[i] = jnp.ones((CNT_W,), jnp.float32)
        return 0
    lax.fori_loop(0, CHUNK, _orow, 0)

    def _crow(i, _):
        cntbuf[i] = jnp.zeros((CNT_W,), jnp.float32)
        return 0
    lax.fori_loop(0, SEG_PER_TILE, _crow, 0)

    base = s * SEG_PER_TILE
    for k in range(SEG_PER_TILE // ZCH):
        pltpu.sync_copy(zbuf, acc_s.at[pl.ds(base + k * ZCH, ZCH)])
    pltpu.sync_copy(cntbuf, cnt_s.at[pl.ds(base, SEG_PER_TILE)])
    plsc.subcore_barrier()

    def _chunk(i, _):
        r0 = s * ROWS_PER_TILE + i * CHUNK
        pltpu.sync_copy(ids_hbm.at[pl.ds(r0, CHUNK)], idx_v)
        pltpu.sync_copy(x_hbm.at[pl.ds(r0, CHUNK), c], rows_v)
        pltpu.sync_copy(rows_v, acc_s.at[idx_v], add=True)

        @pl.when(c == 0)
        def _():
            pltpu.sync_copy(ones_v, cnt_s.at[idx_v], add=True)
        return 0
    lax.fori_loop(0, N_CHUNKS, _chunk, 0)
    plsc.subcore_barrier()

    for k in range(SEG_PER_TILE // ZCH):
        off = base + k * ZCH
        pltpu.sync_copy(acc_s.at[pl.ds(off, ZCH)], zbuf)
        pltpu.sync_copy(zbuf, acc_hbm.at[pl.ds(off, ZCH), c])

    @pl.when(c == 0)
    def _():
        pltpu.sync_copy(cnt_s.at[pl.ds(base, SEG_PER_TILE)], cntbuf)
        pltpu.sync_copy(cntbuf, cnt_hbm.at[pl.ds(base, SEG_PER_TILE)])


@jax.jit
def _segsum_sc(x3, ids):
    mesh = plsc.VectorSubcoreMesh(core_axis_name="c", subcore_axis_name="s")
    fn = pl.kernel(
        _sc_body,
        mesh=mesh,
        compiler_params=pltpu.CompilerParams(use_tc_tiling_on_sc=False),
        out_type=(
            jax.ShapeDtypeStruct((NSEG_PAD, NC, HALF), jnp.float32),
            jax.ShapeDtypeStruct((NSEG_PAD, CNT_W), jnp.float32),
        ),
        scratch_types=[
            pltpu.VMEM_SHARED((NSEG_PAD, HALF), jnp.float32),
            pltpu.VMEM_SHARED((NSEG_PAD, CNT_W), jnp.float32),
            pltpu.VMEM((CHUNK, HALF), jnp.float32),
            pltpu.VMEM((CHUNK,), jnp.int32),
            pltpu.VMEM((CHUNK, CNT_W), jnp.float32),
            pltpu.VMEM((ZCH, HALF), jnp.float32),
            pltpu.VMEM((SEG_PER_TILE, CNT_W), jnp.float32),
        ],
    )
    return fn(x3, ids)


def _mm_body(acc_ref, cnt_ref, wt_ref, b_ref, o_ref):
    o_ref[...] = (
        jnp.dot(acc_ref[...], wt_ref[...], preferred_element_type=jnp.float32)
        + cnt_ref[...][:, 0:1] * b_ref[...]
    )


@jax.jit
def _project_tc(acc, cnt, wt, b2):
    bm = 1000
    return pl.pallas_call(
        _mm_body,
        grid=(N_SEG // bm,),
        in_specs=[
            pl.BlockSpec((bm, D_IN), lambda m: (m, 0)),
            pl.BlockSpec((bm, CNT_W), lambda m: (m, 0)),
            pl.BlockSpec((D_IN, D_OUT), lambda m: (0, 0)),
            pl.BlockSpec((1, D_OUT), lambda m: (0, 0)),
        ],
        out_specs=pl.BlockSpec((bm, D_OUT), lambda m: (m, 0)),
        out_shape=jax.ShapeDtypeStruct((N_SEG, D_OUT), jnp.float32),
    )(acc, cnt, wt, b2)


def kernel(x, signal_belongings, W, b):
    ids = signal_belongings.astype(jnp.int32)
    x3 = x.reshape(N_ROWS, NC, HALF)
    acc3, cnt = _segsum_sc(x3, ids)
    acc = acc3.reshape(NSEG_PAD, D_IN)
    return _project_tc(acc, cnt, W.T, b.reshape(1, D_OUT))
